# baseline (speedup 1.0000x reference)
import functools

import jax
import jax.numpy as jnp
from jax import lax
from jax.experimental import pallas as pl
from jax.experimental.pallas import tpu as pltpu
from jax.experimental.pallas import tpu_sc as plsc

VOCAB = 1000000
D_MODEL = 64
B = 16384
L = 50

_info = plsc.get_sparse_core_info()
NC = _info.num_cores
NS = _info.num_subcores
NW = NC * NS
ROWS_W = B // NW

CB = 8
NCHUNK = ROWS_W // CB
NBUF = 2

_mesh = plsc.VectorSubcoreMesh(core_axis_name="c", subcore_axis_name="s")


@functools.partial(
    pl.kernel,
    mesh=_mesh,
    out_type=jax.ShapeDtypeStruct((B, L, D_MODEL), jnp.float32),
    scratch_types=[
        pltpu.VMEM((ROWS_W, L), jnp.int32),
        pltpu.VMEM((NBUF, CB, L, D_MODEL), jnp.float32),
        pltpu.SemaphoreType.DMA,
        pltpu.SemaphoreType.DMA,
        pltpu.SemaphoreType.DMA,
        pltpu.SemaphoreType.DMA,
    ],
    compiler_params=pltpu.CompilerParams(use_tc_tiling_on_sc=False),
)
def _sc_gather(x_hbm, table_hbm, out_hbm, idx_v, rows_v, g0, g1, o0, o1):
    gsem = (g0, g1)
    osem = (o0, o1)
    wid = lax.axis_index("s") * NC + lax.axis_index("c")
    b0 = wid * ROWS_W

    pltpu.sync_copy(x_hbm.at[pl.ds(pl.multiple_of(b0, 8), ROWS_W)], idx_v)

    def fire_gathers(i, b):
        for j in range(CB):
            pltpu.async_copy(
                table_hbm.at[idx_v.at[i * CB + j]],
                rows_v.at[b].at[j],
                gsem[b],
            )

    def wait_gathers(b):
        pltpu.make_async_copy(
            out_hbm.at[pl.ds(0, CB)], rows_v.at[b], gsem[b]
        ).wait()

    def fire_out(i, b):
        off = pl.multiple_of(b0 + i * CB, CB)
        pltpu.async_copy(rows_v.at[b], out_hbm.at[pl.ds(off, CB)], osem[b])

    def wait_out(b):
        pltpu.make_async_copy(
            out_hbm.at[pl.ds(0, CB)], rows_v.at[b], osem[b]
        ).wait()

    for b in range(NBUF):
        fire_gathers(b, b)

    def body(it, carry):
        for b in range(NBUF):
            i = it * NBUF + b
            wait_gathers(b)
            fire_out(i, b)
            nxt = i + NBUF

            @pl.when(nxt < NCHUNK)
            def _():
                wait_out(b)
                fire_gathers(nxt, b)

        return carry

    lax.fori_loop(0, NCHUNK // NBUF, body, 0)
    for b in range(NBUF):
        wait_out(b)


def kernel(x, weight):
    return _sc_gather(x.astype(jnp.int32), weight)
